# Initial kernel scaffold; baseline (speedup 1.0000x reference)
import jax
import jax.numpy as jnp
from jax.experimental import pallas as pl


def kernel(tokens, p):
    raise NotImplementedError("write your pallas kernel here")


# speedup vs baseline: 322.3234x; 322.3234x over previous
import dataclasses

import jax
import jax.numpy as jnp
import numpy as np
from jax import lax
from jax.experimental import pallas as pl
from jax.experimental.pallas import tpu as pltpu
from jax.experimental.pallas import tpu_sc as plsc

D_MODEL = 1024
END_IDX = 2
NW = 32
INF = np.int32(2**30)


def _sc_build_tree(tokens):
    B, T = tokens.shape
    NCHUNK = T // 16
    mesh = plsc.VectorSubcoreMesh(core_axis_name="c", subcore_axis_name="s")
    cp = pltpu.CompilerParams()
    if "needs_layout_passes" in pltpu.CompilerParams.__dataclass_fields__:
        cp = dataclasses.replace(cp, needs_layout_passes=False)

    @pl.kernel(
        compiler_params=cp,
        out_type=jax.ShapeDtypeStruct((B, T * NW), jnp.int32),
        mesh=mesh,
        scratch_types=[
            pltpu.VMEM((T,), jnp.int32),
            pltpu.VMEM((T + 8,), jnp.int32),
            pltpu.VMEM((T,), jnp.int32),
            pltpu.VMEM((T,), jnp.int32),
            pltpu.VMEM((T * NW,), jnp.int32),
        ],
    )
    def build(tok_hbm, enc_hbm, tok_v, push_v, par_v, pair_v, enc_v):
        wid = lax.axis_index("s") * 2 + lax.axis_index("c")

        @pl.when(wid < B)
        def _():
            b = wid
            pltpu.sync_copy(tok_hbm.at[b], tok_v)
            iota = lax.iota(jnp.int32, 16)
            roll_idx = (iota + 15) & 15
            lane0 = iota == 0
            dnums = lax.GatherDimensionNumbers(
                offset_dims=(), collapsed_slice_dims=(0,), start_index_map=(0,))
            zero16 = jnp.zeros((16,), jnp.int32)

            def roll1(w):
                return lax.gather(
                    w, roll_idx[:, None], dnums, slice_sizes=(1,),
                    mode=lax.GatherScatterMode.PROMISE_IN_BOUNDS)

            push_v[pl.ds(0, 16)] = zero16

            def chunk(i, carry):
                cs, cm = carry
                u = 16 * i + iota
                ld = tok_v[pl.ds(16 * i, 16)]
                ne = ((ld != END_IDX) & (u >= 1)).astype(jnp.int32)
                s = plsc.cumsum(ne) + cs
                bv = jnp.where(u >= 1, 2 * s + 1 - u, INF)
                inc = jnp.minimum(-plsc.cummax(-bv), cm)
                ex = jnp.where(lane0, jnp.full((16,), cm), roll1(inc))
                h = (u - 1) + jnp.minimum(0, ex)
                plsc.store_scatter(push_v, [s], u, mask=ne != 0)
                hidx = jnp.maximum(h >> 1, 0)
                par_v[pl.ds(16 * i, 16)] = plsc.load_gather(push_v, [hidx])
                pair_v[pl.ds(16 * i, 16)] = 1 + (h & 1)
                return (cs + jnp.sum(ne), jnp.minimum(cm, jnp.min(bv)))

            lax.fori_loop(0, NCHUNK, chunk, (np.int32(0), INF))

            enc_v[pl.ds(0, 16)] = zero16
            enc_v[pl.ds(16, 16)] = zero16

            @pl.loop(1, T)
            def step(t):
                t16 = jnp.full((16,), t, jnp.int32)
                par16 = plsc.load_gather(par_v, [t16])
                pair16 = plsc.load_gather(pair_v, [t16])
                base = par16 * NW + iota
                w0 = plsc.load_gather(enc_v, [base])
                w1 = plsc.load_gather(enc_v, [base + 16])
                r0 = roll1(w0)
                r1 = roll1(w1)
                c0 = jnp.where(lane0, pair16, lax.shift_right_logical(r0, 30))
                c1 = lax.shift_right_logical(jnp.where(lane0, r0, r1), 30)
                enc_v[pl.ds(NW * t, 16)] = (w0 << 2) | c0
                enc_v[pl.ds(NW * t + 16, 16)] = (w1 << 2) | c1

            pltpu.sync_copy(enc_v, enc_hbm.at[b])

    return build(tokens)


def _tc_decode(enc_bytes, poly):
    N = enc_bytes.shape[0]
    ROWS = 256
    NB = 4 * NW

    def body(eb_ref, poly_ref, out_ref):
        k = lax.broadcasted_iota(jnp.int32, (ROWS, D_MODEL), 1)
        km = lax.broadcasted_iota(jnp.int32, (NB, D_MODEL), 1)
        mm = lax.broadcasted_iota(jnp.int32, (NB, D_MODEL), 0)
        S = ((km >> 3) == mm).astype(jnp.bfloat16)
        eb = eb_ref[...].astype(jnp.float32).astype(jnp.bfloat16)
        byte = jnp.dot(eb, S, preferred_element_type=jnp.float32).astype(jnp.int32)
        bit = (byte >> (k & 7)) & 1
        out_ref[...] = jnp.where(bit == 1, poly_ref[...], jnp.float32(0.0))

    return pl.pallas_call(
        body,
        grid=(N // ROWS,),
        in_specs=[
            pl.BlockSpec((ROWS, NB), lambda i: (i, 0)),
            pl.BlockSpec((1, D_MODEL), lambda i: (0, 0)),
        ],
        out_specs=pl.BlockSpec((ROWS, D_MODEL), lambda i: (i, 0)),
        out_shape=jax.ShapeDtypeStruct((N, D_MODEL), jnp.float32),
    )(enc_bytes, poly)


def kernel(tokens, p):
    B, T = tokens.shape
    enc = _sc_build_tree(tokens)
    enc_bytes = lax.bitcast_convert_type(enc, jnp.uint8)
    enc_bytes = enc_bytes.reshape(B * T, NW * 4)
    poly = jnp.power(p[0], jnp.arange(D_MODEL, dtype=jnp.float32)).reshape(1, D_MODEL)
    out = _tc_decode(enc_bytes, poly)
    return out.reshape(B, T, D_MODEL)


# speedup vs baseline: 542.6267x; 1.6835x over previous
import dataclasses

import jax
import jax.numpy as jnp
import numpy as np
from jax import lax
from jax.experimental import pallas as pl
from jax.experimental.pallas import tpu as pltpu
from jax.experimental.pallas import tpu_sc as plsc

D_MODEL = 1024
END_IDX = 2
NW = 32
INF = np.int32(2**30)


def _sc_build_tree(tokens):
    B, T = tokens.shape
    NCHUNK = T // 16
    mesh = plsc.VectorSubcoreMesh(core_axis_name="c", subcore_axis_name="s")
    cp = pltpu.CompilerParams()
    if "needs_layout_passes" in pltpu.CompilerParams.__dataclass_fields__:
        cp = dataclasses.replace(cp, needs_layout_passes=False)

    @pl.kernel(
        compiler_params=cp,
        out_type=jax.ShapeDtypeStruct((B, NW * T), jnp.int32),
        mesh=mesh,
        scratch_types=[
            pltpu.VMEM((T,), jnp.int32),
            pltpu.VMEM((T + 8,), jnp.int32),
            pltpu.VMEM((T,), jnp.int32),
            pltpu.VMEM((T,), jnp.int32),
            pltpu.VMEM((NW * T,), jnp.int32),
        ],
    )
    def build(tok_hbm, enc_hbm, tok_v, push_v, par_v, pair_v, enc_v):
        wid = lax.axis_index("s") * 2 + lax.axis_index("c")

        @pl.when(wid < B)
        def _():
            b = wid
            pltpu.sync_copy(tok_hbm.at[b], tok_v)
            iota = lax.iota(jnp.int32, 16)
            roll_idx = (iota + 15) & 15
            lane0 = iota == 0
            dnums = lax.GatherDimensionNumbers(
                offset_dims=(), collapsed_slice_dims=(0,), start_index_map=(0,))
            zero16 = jnp.zeros((16,), jnp.int32)
            plane_lo = T * iota
            plane_hi = T * (iota + 16)

            def roll1(w):
                return lax.gather(
                    w, roll_idx[:, None], dnums, slice_sizes=(1,),
                    mode=lax.GatherScatterMode.PROMISE_IN_BOUNDS)

            push_v[pl.ds(0, 16)] = zero16

            def chunk(i, carry):
                cs, cm = carry
                u = 16 * i + iota
                ld = tok_v[pl.ds(16 * i, 16)]
                ne = ((ld != END_IDX) & (u >= 1)).astype(jnp.int32)
                s = plsc.cumsum(ne) + cs
                bv = jnp.where(u >= 1, 2 * s + 1 - u, INF)
                inc = jnp.minimum(-plsc.cummax(-bv), cm)
                ex = jnp.where(lane0, jnp.full((16,), cm), roll1(inc))
                h = (u - 1) + jnp.minimum(0, ex)
                plsc.store_scatter(push_v, [s], u, mask=ne != 0)
                hidx = jnp.maximum(h >> 1, 0)
                par_v[pl.ds(16 * i, 16)] = plsc.load_gather(push_v, [hidx])
                pair_v[pl.ds(16 * i, 16)] = 1 + (h & 1)
                return (cs + jnp.sum(ne), jnp.minimum(cm, jnp.min(bv)))

            lax.fori_loop(0, NCHUNK, chunk, (np.int32(0), INF))

            plsc.store_scatter(enc_v, [plane_lo], zero16)
            plsc.store_scatter(enc_v, [plane_hi], zero16)

            @pl.loop(1, T)
            def step(t):
                t16 = jnp.full((16,), t, jnp.int32)
                par16 = plsc.load_gather(par_v, [t16])
                pair16 = plsc.load_gather(pair_v, [t16])
                w0 = plsc.load_gather(enc_v, [par16 + plane_lo])
                w1 = plsc.load_gather(enc_v, [par16 + plane_hi])
                r0 = roll1(w0)
                r1 = roll1(w1)
                c0 = jnp.where(lane0, pair16, lax.shift_right_logical(r0, 30))
                c1 = lax.shift_right_logical(jnp.where(lane0, r0, r1), 30)
                plsc.store_scatter(enc_v, [t16 + plane_lo], (w0 << 2) | c0)
                plsc.store_scatter(enc_v, [t16 + plane_hi], (w1 << 2) | c1)

            pltpu.sync_copy(enc_v, enc_hbm.at[b])

    return build(tokens)


def _tc_decode(enc_wt, poly, B, T):
    COLS = 256

    def body(enc_ref, poly_ref, out_ref):
        w = enc_ref[...]
        by = jnp.concatenate(
            [w & 255, (w >> 8) & 255, (w >> 16) & 255, (w >> 24) & 255],
            axis=0)
        by = by.astype(jnp.float32).astype(jnp.bfloat16)
        k = lax.broadcasted_iota(jnp.int32, (COLS, D_MODEL), 1)
        km = lax.broadcasted_iota(jnp.int32, (4 * NW, D_MODEL), 1)
        mm = lax.broadcasted_iota(jnp.int32, (4 * NW, D_MODEL), 0)
        S = ((((km >> 3) & 3) * 32 + (km >> 5)) == mm).astype(jnp.bfloat16)
        byte = lax.dot_general(
            by, S, (((0,), (0,)), ((), ())),
            preferred_element_type=jnp.float32).astype(jnp.int32)
        bit = (byte >> (k & 7)) & 1
        out_ref[...] = jnp.where(bit == 1, poly_ref[...], jnp.float32(0.0))

    return pl.pallas_call(
        body,
        grid=(B, T // COLS),
        in_specs=[
            pl.BlockSpec((NW, COLS), lambda b, c: (b, c)),
            pl.BlockSpec((1, D_MODEL), lambda b, c: (0, 0)),
        ],
        out_specs=pl.BlockSpec(
            (COLS, D_MODEL), lambda b, c: (b * (T // COLS) + c, 0)),
        out_shape=jax.ShapeDtypeStruct((B * T, D_MODEL), jnp.float32),
    )(enc_wt, poly)


def kernel(tokens, p):
    B, T = tokens.shape
    enc = _sc_build_tree(tokens).reshape(B * NW, T)
    poly = jnp.power(p[0], jnp.arange(D_MODEL, dtype=jnp.float32)).reshape(1, D_MODEL)
    out = _tc_decode(enc, poly, B, T)
    return out.reshape(B, T, D_MODEL)


# speedup vs baseline: 785.0935x; 1.4468x over previous
import dataclasses

import jax
import jax.numpy as jnp
import numpy as np
from jax import lax
from jax.experimental import pallas as pl
from jax.experimental.pallas import tpu as pltpu
from jax.experimental.pallas import tpu_sc as plsc

D_MODEL = 1024
END_IDX = 2
NW = 32
INF = np.int32(2**30)


def _sc_build_tree(tokens):
    B, T = tokens.shape
    NCHUNK = T // 16
    mesh = plsc.VectorSubcoreMesh(core_axis_name="c", subcore_axis_name="s")
    cp = pltpu.CompilerParams()
    if "needs_layout_passes" in pltpu.CompilerParams.__dataclass_fields__:
        cp = dataclasses.replace(cp, needs_layout_passes=False)

    @pl.kernel(
        compiler_params=cp,
        out_type=jax.ShapeDtypeStruct((B, T * NW), jnp.int32),
        mesh=mesh,
        scratch_types=[
            pltpu.VMEM((T,), jnp.int32),
            pltpu.VMEM((T + 8,), jnp.int32),
            pltpu.VMEM((T,), jnp.int32),
            pltpu.VMEM((T * NW,), jnp.int32),
        ],
    )
    def build(tok_hbm, enc_hbm, tok_v, push_v, pp_v, enc_v):
        wid = lax.axis_index("s") * 2 + lax.axis_index("c")

        @pl.when(wid < B)
        def _():
            b = wid
            pltpu.sync_copy(tok_hbm.at[b], tok_v)
            iota = lax.iota(jnp.int32, 16)
            roll_idx = (iota + 15) & 15
            lane0 = iota == 0
            dnums = lax.GatherDimensionNumbers(
                offset_dims=(), collapsed_slice_dims=(0,), start_index_map=(0,))
            zero16 = jnp.zeros((16,), jnp.int32)

            def roll1(w):
                return lax.gather(
                    w, roll_idx[:, None], dnums, slice_sizes=(1,),
                    mode=lax.GatherScatterMode.PROMISE_IN_BOUNDS)

            push_v[pl.ds(0, 16)] = zero16

            def chunk(i, carry):
                cs, cm = carry
                u = 16 * i + iota
                ld = tok_v[pl.ds(16 * i, 16)]
                ne = ((ld != END_IDX) & (u >= 1)).astype(jnp.int32)
                s = plsc.cumsum(ne) + cs
                bv = jnp.where(u >= 1, 2 * s + 1 - u, INF)
                inc = jnp.minimum(-plsc.cummax(-bv), cm)
                ex = jnp.where(lane0, jnp.full((16,), cm), roll1(inc))
                h = (u - 1) + jnp.minimum(0, ex)
                plsc.store_scatter(push_v, [s], u, mask=ne != 0)
                hidx = jnp.maximum(h >> 1, 0)
                par = plsc.load_gather(push_v, [hidx])
                pp_v[pl.ds(16 * i, 16)] = 4 * par + 1 + (h & 1)
                return (cs + jnp.sum(ne), jnp.minimum(cm, jnp.min(bv)))

            lax.fori_loop(0, NCHUNK, chunk, (np.int32(0), INF))

            enc_v[pl.ds(0, 16)] = zero16
            enc_v[pl.ds(16, 16)] = zero16

            @pl.loop(1, T)
            def step(t):
                t16 = jnp.full((16,), t, jnp.int32)
                pp16 = plsc.load_gather(pp_v, [t16])
                pair16 = pp16 & 3
                par16 = pp16 >> 2
                pbase = ((par16 >> 2) << 7) + ((par16 & 3) << 5) + iota
                w0 = plsc.load_gather(enc_v, [pbase])
                w1 = plsc.load_gather(enc_v, [pbase + 16])
                r0 = roll1(w0)
                r1 = roll1(w1)
                c0 = jnp.where(lane0, pair16, lax.shift_right_logical(r0, 30))
                c1 = lax.shift_right_logical(jnp.where(lane0, r0, r1), 30)
                toff = ((t >> 2) << 7) + ((t & 3) << 5)
                enc_v[pl.ds(toff, 16)] = (w0 << 2) | c0
                enc_v[pl.ds(toff + 16, 16)] = (w1 << 2) | c1

            pltpu.sync_copy(enc_v, enc_hbm.at[b])

    return build(tokens)


def _selector_consts():
    l = np.arange(512)[:, None]
    c = np.arange(4096)[None, :]
    lbi, lg, lw = l >> 7, (l >> 5) & 3, l & 31
    cg, ck = c >> 10, c & 1023
    s1 = (lg == cg) & (lw == (ck >> 5)) & (lbi == ((ck >> 3) & 3))
    tau = np.arange(256)
    pm = np.zeros((256, 256), np.bool_)
    pm[tau, 64 * (tau & 3) + (tau >> 2)] = True
    return (jnp.asarray(s1.astype(np.float32), dtype=jnp.bfloat16),
            jnp.asarray(pm.astype(np.float32), dtype=jnp.bfloat16))


def _tc_decode(enc_g, s1, pm, poly):
    NROW = enc_g.shape[0]
    ROWS = 64

    def body(enc_ref, s1_ref, pm_ref, poly_ref, out_ref):
        w = enc_ref[...]
        by = jnp.concatenate(
            [w & 255, (w >> 8) & 255, (w >> 16) & 255, (w >> 24) & 255],
            axis=1)
        by = by.astype(jnp.float32).astype(jnp.bfloat16)
        m1 = jnp.dot(by, s1_ref[...],
                     preferred_element_type=jnp.float32)
        z = jnp.concatenate(
            [m1[:, 0:1024], m1[:, 1024:2048], m1[:, 2048:3072],
             m1[:, 3072:4096]], axis=0)
        z = z.astype(jnp.bfloat16)
        byte = jnp.dot(pm_ref[...], z,
                       preferred_element_type=jnp.float32).astype(jnp.int32)
        k = lax.broadcasted_iota(jnp.int32, (4 * ROWS, D_MODEL), 1)
        bit = (byte >> (k & 7)) & 1
        out_ref[...] = jnp.where(bit == 1, poly_ref[...], jnp.float32(0.0))

    return pl.pallas_call(
        body,
        grid=(NROW // ROWS,),
        in_specs=[
            pl.BlockSpec((ROWS, 128), lambda i: (i, 0)),
            pl.BlockSpec((512, 4096), lambda i: (0, 0)),
            pl.BlockSpec((256, 256), lambda i: (0, 0)),
            pl.BlockSpec((1, D_MODEL), lambda i: (0, 0)),
        ],
        out_specs=pl.BlockSpec((4 * ROWS, D_MODEL), lambda i: (i, 0)),
        out_shape=jax.ShapeDtypeStruct((4 * NROW, D_MODEL), jnp.float32),
    )(enc_g, s1, pm, poly)


def kernel(tokens, p):
    B, T = tokens.shape
    enc = _sc_build_tree(tokens)
    enc_g = enc.reshape(B * T // 4, 128)
    s1, pm = _selector_consts()
    poly = jnp.power(p[0], jnp.arange(D_MODEL, dtype=jnp.float32)).reshape(1, D_MODEL)
    out = _tc_decode(enc_g, s1, pm, poly)
    return out.reshape(B, T, D_MODEL)


# speedup vs baseline: 882.9563x; 1.1247x over previous
import dataclasses

import jax
import jax.numpy as jnp
import numpy as np
from jax import lax
from jax.experimental import pallas as pl
from jax.experimental.pallas import tpu as pltpu
from jax.experimental.pallas import tpu_sc as plsc

D_MODEL = 1024
END_IDX = 2
NW = 32
INF = np.int32(2**30)


def _sc_build_tree(tokens):
    B, T = tokens.shape
    NCHUNK = T // 16
    mesh = plsc.VectorSubcoreMesh(core_axis_name="c", subcore_axis_name="s")
    cp = pltpu.CompilerParams()
    if "needs_layout_passes" in pltpu.CompilerParams.__dataclass_fields__:
        cp = dataclasses.replace(cp, needs_layout_passes=False)

    @pl.kernel(
        compiler_params=cp,
        out_type=jax.ShapeDtypeStruct((B, T * NW), jnp.int32),
        mesh=mesh,
        scratch_types=[
            pltpu.VMEM((T,), jnp.int32),
            pltpu.VMEM((T + 8,), jnp.int32),
            pltpu.VMEM((T,), jnp.int32),
            pltpu.VMEM((T * NW,), jnp.int32),
        ],
    )
    def build(tok_hbm, enc_hbm, tok_v, push_v, pp_v, enc_v):
        wid = lax.axis_index("s") * 2 + lax.axis_index("c")

        @pl.when(wid < B)
        def _():
            b = wid
            pltpu.sync_copy(tok_hbm.at[b], tok_v)
            iota = lax.iota(jnp.int32, 16)
            roll_idx = (iota + 15) & 15
            lane0 = iota == 0
            dnums = lax.GatherDimensionNumbers(
                offset_dims=(), collapsed_slice_dims=(0,), start_index_map=(0,))
            zero16 = jnp.zeros((16,), jnp.int32)

            def roll1(w):
                return lax.gather(
                    w, roll_idx[:, None], dnums, slice_sizes=(1,),
                    mode=lax.GatherScatterMode.PROMISE_IN_BOUNDS)

            push_v[pl.ds(0, 16)] = zero16

            def chunk(i, carry):
                cs, cm = carry
                u = 16 * i + iota
                ld = tok_v[pl.ds(16 * i, 16)]
                ne = ((ld != END_IDX) & (u >= 1)).astype(jnp.int32)
                s = plsc.cumsum(ne) + cs
                bv = jnp.where(u >= 1, 2 * s + 1 - u, INF)
                inc = jnp.minimum(-plsc.cummax(-bv), cm)
                ex = jnp.where(lane0, jnp.full((16,), cm), roll1(inc))
                h = (u - 1) + jnp.minimum(0, ex)
                plsc.store_scatter(push_v, [s], u, mask=ne != 0)
                hidx = jnp.maximum(h >> 1, 0)
                par = plsc.load_gather(push_v, [hidx])
                pp_v[pl.ds(16 * i, 16)] = 4 * par + 1 + (h & 1)
                return (cs + jnp.sum(ne), jnp.minimum(cm, jnp.min(bv)))

            lax.fori_loop(0, NCHUNK, chunk, (np.int32(0), INF))

            enc_v[pl.ds(0, 16)] = zero16
            enc_v[pl.ds(16, 16)] = zero16

            @pl.loop(1, T)
            def step(t):
                t16 = jnp.full((16,), t, jnp.int32)
                pp16 = plsc.load_gather(pp_v, [t16])
                pair16 = pp16 & 3
                par16 = pp16 >> 2
                pbase = (((par16 >> 8) << 13) + ((par16 & 63) << 7)
                         + (((par16 >> 6) & 3) << 5) + iota)
                w0 = plsc.load_gather(enc_v, [pbase])
                w1 = plsc.load_gather(enc_v, [pbase + 16])
                r0 = roll1(w0)
                r1 = roll1(w1)
                c0 = jnp.where(lane0, pair16, lax.shift_right_logical(r0, 30))
                c1 = lax.shift_right_logical(jnp.where(lane0, r0, r1), 30)
                toff = (((t >> 8) << 13) + ((t & 63) << 7)
                        + (((t >> 6) & 3) << 5))
                enc_v[pl.ds(toff, 16)] = (w0 << 2) | c0
                enc_v[pl.ds(toff + 16, 16)] = (w1 << 2) | c1

            pltpu.sync_copy(enc_v, enc_hbm.at[b])

    return build(tokens)


def _selector_const():
    m = np.arange(128)[:, None]
    k = np.arange(D_MODEL)[None, :]
    s = ((m & 31) == (k >> 5)) & ((m >> 5) == ((k >> 3) & 3))
    return jnp.asarray(s.astype(np.float32), dtype=jnp.bfloat16)


def _tc_decode(enc_g, sel, poly):
    NROW = enc_g.shape[0]
    ROWS = 64

    def body(enc_ref, sel_ref, poly_ref, out_ref):
        w = enc_ref[...]
        parts = []
        for g in range(4):
            wg = w[:, 32 * g:32 * (g + 1)]
            parts.append(jnp.concatenate(
                [wg & 255, (wg >> 8) & 255, (wg >> 16) & 255,
                 (wg >> 24) & 255], axis=1))
        by = jnp.concatenate(parts, axis=0)
        by = by.astype(jnp.float32).astype(jnp.bfloat16)
        byte = jnp.dot(by, sel_ref[...],
                       preferred_element_type=jnp.float32).astype(jnp.int32)
        k = lax.broadcasted_iota(jnp.int32, (4 * ROWS, D_MODEL), 1)
        bit = (byte >> (k & 7)) & 1
        out_ref[...] = jnp.where(bit == 1, poly_ref[...], jnp.float32(0.0))

    return pl.pallas_call(
        body,
        grid=(NROW // ROWS,),
        in_specs=[
            pl.BlockSpec((ROWS, 128), lambda i: (i, 0)),
            pl.BlockSpec((128, D_MODEL), lambda i: (0, 0)),
            pl.BlockSpec((1, D_MODEL), lambda i: (0, 0)),
        ],
        out_specs=pl.BlockSpec((4 * ROWS, D_MODEL), lambda i: (i, 0)),
        out_shape=jax.ShapeDtypeStruct((4 * NROW, D_MODEL), jnp.float32),
    )(enc_g, sel, poly)


def kernel(tokens, p):
    B, T = tokens.shape
    enc = _sc_build_tree(tokens)
    enc_g = enc.reshape(B * T // 4, 128)
    sel = _selector_const()
    poly = jnp.power(p[0], jnp.arange(D_MODEL, dtype=jnp.float32)).reshape(1, D_MODEL)
    out = _tc_decode(enc_g, sel, poly)
    return out.reshape(B, T, D_MODEL)


# speedup vs baseline: 1084.2995x; 1.2280x over previous
import dataclasses

import jax
import jax.numpy as jnp
import numpy as np
from jax import lax
from jax.experimental import pallas as pl
from jax.experimental.pallas import tpu as pltpu
from jax.experimental.pallas import tpu_sc as plsc

D_MODEL = 1024
END_IDX = 2
NW = 32
INF = np.int32(2**30)


def _sc_build_tree(tokens):
    B, T = tokens.shape
    NCHUNK = T // 16
    mesh = plsc.VectorSubcoreMesh(core_axis_name="c", subcore_axis_name="s")
    cp = pltpu.CompilerParams()
    if "needs_layout_passes" in pltpu.CompilerParams.__dataclass_fields__:
        cp = dataclasses.replace(cp, needs_layout_passes=False)

    @pl.kernel(
        compiler_params=cp,
        out_type=jax.ShapeDtypeStruct((B, T * NW), jnp.int32),
        mesh=mesh,
        scratch_types=[
            pltpu.VMEM((T,), jnp.int32),
            pltpu.VMEM((T + 8,), jnp.int32),
            pltpu.VMEM((T,), jnp.int32),
            pltpu.VMEM((T * NW,), jnp.int32),
        ],
    )
    def build(tok_hbm, enc_hbm, tok_v, push_v, pp_v, enc_v):
        wid = lax.axis_index("s") * 2 + lax.axis_index("c")

        @pl.when(wid < B)
        def _():
            b = wid
            pltpu.sync_copy(tok_hbm.at[b], tok_v)
            iota = lax.iota(jnp.int32, 16)
            roll_idx = (iota + 15) & 15
            lane0 = iota == 0
            dnums = lax.GatherDimensionNumbers(
                offset_dims=(), collapsed_slice_dims=(0,), start_index_map=(0,))
            zero16 = jnp.zeros((16,), jnp.int32)

            def roll1(w):
                return lax.gather(
                    w, roll_idx[:, None], dnums, slice_sizes=(1,),
                    mode=lax.GatherScatterMode.PROMISE_IN_BOUNDS)

            push_v[pl.ds(0, 16)] = zero16

            def chunk(i, carry):
                cs, cm = carry
                u = 16 * i + iota
                ld = tok_v[pl.ds(16 * i, 16)]
                ne = ((ld != END_IDX) & (u >= 1)).astype(jnp.int32)
                s = plsc.cumsum(ne) + cs
                bv = jnp.where(u >= 1, 2 * s + 1 - u, INF)
                inc = jnp.minimum(-plsc.cummax(-bv), cm)
                ex = jnp.where(lane0, jnp.full((16,), cm), roll1(inc))
                h = (u - 1) + jnp.minimum(0, ex)
                plsc.store_scatter(push_v, [s], u, mask=ne != 0)
                hidx = jnp.maximum(h >> 1, 0)
                par = plsc.load_gather(push_v, [hidx])
                pp_v[pl.ds(16 * i, 16)] = 4 * par + 1 + (h & 1)
                return (cs + jnp.sum(ne), jnp.minimum(cm, jnp.min(bv)))

            lax.fori_loop(0, NCHUNK, chunk, (np.int32(0), INF))

            enc_v[pl.ds(0, 16)] = zero16
            enc_v[pl.ds(16, 16)] = zero16
            jconst = [jnp.full((16,), j, jnp.int32) for j in range(16)]

            def lane_bcast(vec, j):
                return lax.gather(
                    vec, jconst[j][:, None], dnums, slice_sizes=(1,),
                    mode=lax.GatherScatterMode.PROMISE_IN_BOUNDS)

            def step(t, pp16):
                pair16 = pp16 & 3
                par16 = pp16 >> 2
                pbase = (((par16 >> 8) << 13) + ((par16 & 63) << 7)
                         + (((par16 >> 6) & 3) << 5) + iota)
                w0 = plsc.load_gather(enc_v, [pbase])
                w1 = plsc.load_gather(enc_v, [pbase + 16])
                r0 = roll1(w0)
                r1 = roll1(w1)
                c0 = jnp.where(lane0, pair16, lax.shift_right_logical(r0, 30))
                c1 = lax.shift_right_logical(jnp.where(lane0, r0, r1), 30)
                toff = (((t >> 8) << 13) + ((t & 63) << 7)
                        + (((t >> 6) & 3) << 5))
                enc_v[pl.ds(toff, 16)] = (w0 << 2) | c0
                enc_v[pl.ds(toff + 16, 16)] = (w1 << 2) | c1

            pp_c0 = pp_v[pl.ds(0, 16)]
            for j in range(1, 16):
                step(j, lane_bcast(pp_c0, j))

            @pl.loop(1, NCHUNK)
            def outer(i):
                pp_chunk = pp_v[pl.ds(16 * i, 16)]
                for j in range(16):
                    step(16 * i + j, lane_bcast(pp_chunk, j))

            pltpu.sync_copy(enc_v, enc_hbm.at[b])

    return build(tokens)


def _selector_const():
    m = np.arange(128)[:, None]
    k = np.arange(D_MODEL)[None, :]
    s = ((m & 31) == (k >> 5)) & ((m >> 5) == ((k >> 3) & 3))
    return jnp.asarray(s.astype(np.float32), dtype=jnp.bfloat16)


def _tc_decode(enc_g, sel, poly):
    NROW = enc_g.shape[0]
    ROWS = 64

    def body(enc_ref, sel_ref, poly_ref, out_ref):
        w = enc_ref[...]
        parts = []
        for g in range(4):
            wg = w[:, 32 * g:32 * (g + 1)]
            parts.append(jnp.concatenate(
                [wg & 255, (wg >> 8) & 255, (wg >> 16) & 255,
                 (wg >> 24) & 255], axis=1))
        by = jnp.concatenate(parts, axis=0)
        by = by.astype(jnp.float32).astype(jnp.bfloat16)
        byte = jnp.dot(by, sel_ref[...],
                       preferred_element_type=jnp.float32).astype(jnp.int32)
        k = lax.broadcasted_iota(jnp.int32, (4 * ROWS, D_MODEL), 1)
        bit = (byte >> (k & 7)) & 1
        out_ref[...] = jnp.where(bit == 1, poly_ref[...], jnp.float32(0.0))

    return pl.pallas_call(
        body,
        grid=(NROW // ROWS,),
        in_specs=[
            pl.BlockSpec((ROWS, 128), lambda i: (i, 0)),
            pl.BlockSpec((128, D_MODEL), lambda i: (0, 0)),
            pl.BlockSpec((1, D_MODEL), lambda i: (0, 0)),
        ],
        out_specs=pl.BlockSpec((4 * ROWS, D_MODEL), lambda i: (i, 0)),
        out_shape=jax.ShapeDtypeStruct((4 * NROW, D_MODEL), jnp.float32),
    )(enc_g, sel, poly)


def kernel(tokens, p):
    B, T = tokens.shape
    enc = _sc_build_tree(tokens)
    enc_g = enc.reshape(B * T // 4, 128)
    sel = _selector_const()
    poly = jnp.power(p[0], jnp.arange(D_MODEL, dtype=jnp.float32)).reshape(1, D_MODEL)
    out = _tc_decode(enc_g, sel, poly)
    return out.reshape(B, T, D_MODEL)


# speedup vs baseline: 1124.4183x; 1.0370x over previous
import dataclasses

import jax
import jax.numpy as jnp
import numpy as np
from jax import lax
from jax.experimental import pallas as pl
from jax.experimental.pallas import tpu as pltpu
from jax.experimental.pallas import tpu_sc as plsc

D_MODEL = 1024
END_IDX = 2
NW = 32
INF = np.int32(2**30)


def _sc_build_tree(tokens):
    B, T = tokens.shape
    NCHUNK = T // 16
    mesh = plsc.VectorSubcoreMesh(core_axis_name="c", subcore_axis_name="s")
    cp = pltpu.CompilerParams()
    if "needs_layout_passes" in pltpu.CompilerParams.__dataclass_fields__:
        cp = dataclasses.replace(cp, needs_layout_passes=False)

    @pl.kernel(
        compiler_params=cp,
        out_type=jax.ShapeDtypeStruct((B, T // 4, 128), jnp.int32),
        mesh=mesh,
        scratch_types=[
            pltpu.VMEM((T,), jnp.int32),
            pltpu.VMEM((T + 8,), jnp.int32),
            pltpu.VMEM((T,), jnp.int32),
            pltpu.VMEM((T // 4, 128), jnp.int32),
        ],
    )
    def build(tok_hbm, enc_hbm, tok_v, push_v, pp_v, enc_v):
        wid = lax.axis_index("s") * 2 + lax.axis_index("c")

        @pl.when(wid < B)
        def _():
            b = wid
            pltpu.sync_copy(tok_hbm.at[b], tok_v)
            iota = lax.iota(jnp.int32, 16)
            roll_idx = (iota + 15) & 15
            lane0 = iota == 0
            dnums = lax.GatherDimensionNumbers(
                offset_dims=(), collapsed_slice_dims=(0,), start_index_map=(0,))
            zero16 = jnp.zeros((16,), jnp.int32)

            def roll1(w):
                return lax.gather(
                    w, roll_idx[:, None], dnums, slice_sizes=(1,),
                    mode=lax.GatherScatterMode.PROMISE_IN_BOUNDS)

            push_v[pl.ds(0, 16)] = zero16

            def chunk(i, carry):
                cs, cm = carry
                u = 16 * i + iota
                ld = tok_v[pl.ds(16 * i, 16)]
                ne = ((ld != END_IDX) & (u >= 1)).astype(jnp.int32)
                s = plsc.cumsum(ne) + cs
                bv = jnp.where(u >= 1, 2 * s + 1 - u, INF)
                inc = jnp.minimum(-plsc.cummax(-bv), cm)
                ex = jnp.where(lane0, jnp.full((16,), cm), roll1(inc))
                h = (u - 1) + jnp.minimum(0, ex)
                plsc.store_scatter(push_v, [s], u, mask=ne != 0)
                hidx = jnp.maximum(h >> 1, 0)
                par = plsc.load_gather(push_v, [hidx])
                pp_v[pl.ds(16 * i, 16)] = 4 * par + 1 + (h & 1)
                return (cs + jnp.sum(ne), jnp.minimum(cm, jnp.min(bv)))

            lax.fori_loop(0, NCHUNK, chunk, (np.int32(0), INF))

            enc_v[0, pl.ds(0, 16)] = zero16
            enc_v[0, pl.ds(16, 16)] = zero16
            jconst = [jnp.full((16,), j, jnp.int32) for j in range(16)]

            def lane_bcast(vec, j):
                return lax.gather(
                    vec, jconst[j][:, None], dnums, slice_sizes=(1,),
                    mode=lax.GatherScatterMode.PROMISE_IN_BOUNDS)

            def step(t, pp16):
                pair16 = pp16 & 3
                par16 = pp16 >> 2
                prow = ((par16 >> 8) << 6) + (par16 & 63)
                plane = (((par16 >> 6) & 3) << 5) + iota
                w0 = plsc.load_gather(enc_v, [prow, plane])
                w1 = plsc.load_gather(enc_v, [prow, plane + 16])
                r0 = roll1(w0)
                r1 = roll1(w1)
                c0 = jnp.where(lane0, pair16, lax.shift_right_logical(r0, 30))
                c1 = lax.shift_right_logical(jnp.where(lane0, r0, r1), 30)
                trow = ((t >> 8) << 6) + (t & 63)
                tlane = ((t >> 6) & 3) << 5
                enc_v[trow, pl.ds(tlane, 16)] = (w0 << 2) | c0
                enc_v[trow, pl.ds(tlane + 16, 16)] = (w1 << 2) | c1

            pp_c0 = pp_v[pl.ds(0, 16)]
            for j in range(1, 16):
                step(j, lane_bcast(pp_c0, j))

            @pl.loop(1, NCHUNK)
            def outer(i):
                pp_chunk = pp_v[pl.ds(16 * i, 16)]
                for j in range(16):
                    step(16 * i + j, lane_bcast(pp_chunk, j))

            pltpu.sync_copy(enc_v, enc_hbm.at[b])

    return build(tokens)


def _selector_const():
    m = np.arange(128)[:, None]
    k = np.arange(D_MODEL)[None, :]
    s = ((m & 31) == (k >> 5)) & ((m >> 5) == ((k >> 3) & 3))
    return jnp.asarray(s.astype(np.float32), dtype=jnp.bfloat16)


def _tc_decode(enc_g, sel, mask, poly):
    NROW = enc_g.shape[0]
    ROWS = 64

    def body(enc_ref, sel_ref, mask_ref, poly_ref, out_ref):
        w = enc_ref[...]
        parts = []
        for g in range(4):
            wg = w[:, 32 * g:32 * (g + 1)]
            parts.append(jnp.concatenate(
                [wg & 255, (wg >> 8) & 255, (wg >> 16) & 255,
                 (wg >> 24) & 255], axis=1))
        by = jnp.concatenate(parts, axis=0)
        by = by.astype(jnp.float32).astype(jnp.bfloat16)
        byte = jnp.dot(by, sel_ref[...],
                       preferred_element_type=jnp.float32).astype(jnp.int32)
        out_ref[...] = jnp.where((byte & mask_ref[...]) != 0,
                                 poly_ref[...], jnp.float32(0.0))

    return pl.pallas_call(
        body,
        grid=(NROW // ROWS,),
        in_specs=[
            pl.BlockSpec((ROWS, 128), lambda i: (i, 0)),
            pl.BlockSpec((128, D_MODEL), lambda i: (0, 0)),
            pl.BlockSpec((1, D_MODEL), lambda i: (0, 0)),
            pl.BlockSpec((1, D_MODEL), lambda i: (0, 0)),
        ],
        out_specs=pl.BlockSpec((4 * ROWS, D_MODEL), lambda i: (i, 0)),
        out_shape=jax.ShapeDtypeStruct((4 * NROW, D_MODEL), jnp.float32),
    )(enc_g, sel, mask, poly)


def kernel(tokens, p):
    B, T = tokens.shape
    enc = _sc_build_tree(tokens)
    enc_g = enc.reshape(B * T // 4, 128)
    sel = _selector_const()
    k = np.arange(D_MODEL)
    mask = jnp.asarray((1 << (k & 7)).astype(np.int32)).reshape(1, D_MODEL)
    poly = jnp.power(p[0], jnp.arange(D_MODEL, dtype=jnp.float32)).reshape(1, D_MODEL)
    out = _tc_decode(enc_g, sel, mask, poly)
    return out.reshape(B, T, D_MODEL)
